# baseline (speedup 1.0000x reference)
import functools

import jax
import jax.numpy as jnp
from jax import lax
from jax.experimental import pallas as pl
from jax.experimental.pallas import tpu as pltpu, tpu_sc as plsc

T_MAX = 1000
BETA_MIN = 0.0001
BETA_MAX = 0.02

_L = 16
_NC = 1
_NS = 16
_NW = _NC * _NS


def _sc_kernel(B):
    b_per_w = B // _NW
    mesh = plsc.VectorSubcoreMesh(core_axis_name="c", subcore_axis_name="s", num_cores=_NC)

    @functools.partial(
        pl.kernel,
        mesh=mesh,
        out_type=(
            jax.ShapeDtypeStruct((B,), jnp.float32),
            jax.ShapeDtypeStruct((B,), jnp.float32),
        ),
        scratch_types=[
            pltpu.VMEM((b_per_w,), jnp.int32),
            pltpu.VMEM((T_MAX,), jnp.float32),
            pltpu.VMEM((b_per_w,), jnp.float32),
            pltpu.VMEM((b_per_w,), jnp.float32),
            pltpu.SemaphoreType.DMA,
            pltpu.SemaphoreType.DMA,
            pltpu.SemaphoreType.DMA,
            pltpu.SemaphoreType.DMA,
        ],
        compiler_params=pltpu.CompilerParams(
            needs_layout_passes=False,
            skip_device_barrier=True,
            disable_bounds_checks=True,
            disable_semaphore_checks=True,
        ),
    )
    def k(t_hbm, abars_hbm, abars_t_hbm, betas_hbm, idx_v, tab_v, oa_v, ob_v,
          sem0, sem1, sem2, sem3):
        wid = lax.axis_index("s") * _NC + lax.axis_index("c") if _NC > 1 else lax.axis_index("s")
        base = wid * b_per_w
        H = b_per_w // 2
        cp_idx = pltpu.async_copy(t_hbm.at[pl.ds(base, b_per_w)], idx_v, sem0)
        cp_tab = pltpu.async_copy(abars_hbm, tab_v, sem1)
        scale = jnp.float32((BETA_MAX - BETA_MIN) / T_MAX)
        bmin = jnp.float32(BETA_MIN)
        cp_idx.wait()
        cp_tab.wait()

        @plsc.parallel_loop(0, H, _L, unroll=4)
        def body0(o):
            tv = idx_v[pl.ds(o, _L)]
            oa_v[pl.ds(o, _L)] = plsc.load_gather(tab_v, [tv])
            ob_v[pl.ds(o, _L)] = tv.astype(jnp.float32) * scale + bmin

        cp_a0 = pltpu.async_copy(oa_v.at[pl.ds(0, H)], abars_t_hbm.at[pl.ds(base, H)], sem0)
        cp_b0 = pltpu.async_copy(ob_v.at[pl.ds(0, H)], betas_hbm.at[pl.ds(base, H)], sem1)

        @plsc.parallel_loop(H, b_per_w, _L, unroll=4)
        def body1(o):
            tv = idx_v[pl.ds(o, _L)]
            oa_v[pl.ds(o, _L)] = plsc.load_gather(tab_v, [tv])
            ob_v[pl.ds(o, _L)] = tv.astype(jnp.float32) * scale + bmin

        cp_a1 = pltpu.async_copy(oa_v.at[pl.ds(H, H)], abars_t_hbm.at[pl.ds(base + H, H)], sem2)
        cp_b1 = pltpu.async_copy(ob_v.at[pl.ds(H, H)], betas_hbm.at[pl.ds(base + H, H)], sem3)
        cp_a0.wait()
        cp_b0.wait()
        cp_a1.wait()
        cp_b1.wait()

    return k


def kernel(t, abars):
    B = t.shape[0]
    abars_t, betas = _sc_kernel(B)(t, abars)
    return (abars_t, betas)
